# baseline (speedup 1.0000x reference)
import functools

import jax
import jax.numpy as jnp
from jax import lax
from jax.experimental import pallas as pl
from jax.experimental.pallas import tpu as pltpu
from jax.experimental.pallas import tpu_sc as plsc

N, E, D = 10000, 320000, 128
R3 = 96
R = 32

NC, NS = 2, 16
NW = NC * NS
EPW = E // NW
CH = 80
CPW = EPW // CH
KG = 5
NPAD = 10240
NPT = NPAD // NS


def _silu(x):
    return x * jax.nn.sigmoid(x)


def _a_body(h_ref, a1_ref, a2_ref, hd_ref, hs_ref):
    h = h_ref[...]
    hd_ref[...] = jnp.dot(h, a1_ref[...], preferred_element_type=jnp.float32)
    hs_ref[...] = jnp.dot(h, a2_ref[...], preferred_element_type=jnp.float32)


NG = 5
CPG = CPW // NG


def _gather_body(hd, hs, dst4, src4, gd, gs,
                 idxd_v, idxs_v, bufd, bufs, semg, semw):
    c = lax.axis_index("c")
    s = lax.axis_index("s")
    w = s * NC + c

    def group(g, carry0):
        pltpu.sync_copy(dst4.at[w, g], idxd_v)
        pltpu.sync_copy(src4.at[w, g], idxs_v)

        def super_iter(i, carry):
            c0 = i * KG
            cps = []
            for k in range(KG):
                cps.append(pltpu.async_copy(
                    hd.at[idxd_v.at[c0 + k]], bufd.at[pl.ds(k * CH, CH)], semg))
                cps.append(pltpu.async_copy(
                    hs.at[idxs_v.at[c0 + k]], bufs.at[pl.ds(k * CH, CH)], semg))
            for cp in cps:
                cp.wait()
            eb = ((w * NG + g) * CPG + c0) * CH
            cpo1 = pltpu.async_copy(bufd, gd.at[pl.ds(eb, KG * CH)], semw)
            cpo2 = pltpu.async_copy(bufs, gs.at[pl.ds(eb, KG * CH)], semw)
            cpo1.wait()
            cpo2.wait()
            return carry

        lax.fori_loop(0, CPG // KG, super_iter, 0)
        return carry0

    lax.fori_loop(0, NG, group, 0)


def _c_body(ef_ref, gd_ref, gs_ref, a3_ref, bm_ref, bmsg_ref,
            wea_ref, web_ref, bwe_ref, x2a_ref, lng_ref, lnb_ref,
            mc_ref, eo_ref):
    e = ef_ref[...]
    w = jnp.dot(jnp.dot(e, wea_ref[...], preferred_element_type=jnp.float32),
                web_ref[...], preferred_element_type=jnp.float32) + bwe_ref[...]
    t = gd_ref[...] + gs_ref[...] + jnp.dot(
        e, a3_ref[...], preferred_element_type=jnp.float32)
    u = jnp.dot(t, bm_ref[...], preferred_element_type=jnp.float32) + bmsg_ref[...]
    m = _silu(u) * w
    mc_ref[...] = jnp.dot(m, x2a_ref[...], preferred_element_type=jnp.float32)
    v = e + w
    mu = jnp.mean(v, axis=-1, keepdims=True)
    var = jnp.mean((v - mu) ** 2, axis=-1, keepdims=True)
    vn = (v - mu) / jnp.sqrt(var + 1e-5) * lng_ref[...] + lnb_ref[...]
    eo_ref[...] = _silu(vn)


def _scatter_body(mc, dst2, zeros, out,
                  idx_v, rows, zbuf, acc, semr, sema):
    c = lax.axis_index("c")
    s = lax.axis_index("s")
    w = s * NC + c
    row0 = w * CPW
    pltpu.sync_copy(zeros.at[pl.ds(s * NPT, NPT)], zbuf)
    pltpu.sync_copy(zbuf, acc.at[pl.ds(s * NPT, NPT)])
    plsc.subcore_barrier()
    pltpu.sync_copy(dst2.at[w], idx_v)

    KS = 25

    def super_iter(i, carry):
        c0 = i * KS
        eb = (row0 + c0) * CH
        pltpu.sync_copy(mc.at[pl.ds(eb, KS * CH)], rows)
        cps = []
        for k in range(KS):
            cps.append(pltpu.async_copy(
                rows.at[pl.ds(k * CH, CH)], acc.at[idx_v.at[c0 + k]],
                sema, add=True))
        for cp in cps:
            cp.wait()
        return carry

    lax.fori_loop(0, CPW // KS, super_iter, 0)
    plsc.subcore_barrier()
    pltpu.sync_copy(acc.at[pl.ds(s * NPT, NPT)], zbuf)
    pltpu.sync_copy(zbuf, out.at[c, pl.ds(s * NPT, NPT)])


def _e_body(h_ref, p0_ref, p1_ref, x2a_ref, x2b_ref, bx2_ref,
            ra1_ref, rb1_ref, rc1_ref, ra2_ref, rb2_ref, rc2_ref,
            lng_ref, lnb_ref, x_ref):
    hh = h_ref[...]
    z = jnp.dot(hh, x2a_ref[...], preferred_element_type=jnp.float32)
    z = z + p0_ref[...] + p1_ref[...]
    u = jnp.dot(z, x2b_ref[...], preferred_element_type=jnp.float32) + bx2_ref[...]
    x = _silu(u)
    for k in range(3):
        y = _silu(jnp.dot(jnp.dot(x, ra1_ref[k], preferred_element_type=jnp.float32),
                          rb1_ref[k], preferred_element_type=jnp.float32)
                  + rc1_ref[k])
        y = _silu(jnp.dot(jnp.dot(y, ra2_ref[k], preferred_element_type=jnp.float32),
                          rb2_ref[k], preferred_element_type=jnp.float32)
                  + rc2_ref[k])
        x = x + y
    mu = jnp.mean(x, axis=-1, keepdims=True)
    var = jnp.mean((x - mu) ** 2, axis=-1, keepdims=True)
    xn = (x - mu) / jnp.sqrt(var + 1e-5) * lng_ref[...] + lnb_ref[...]
    x_ref[...] = hh + _silu(xn)


def kernel(node_feats, edge_feats, edge_index, params):
    f32 = jnp.float32
    h = node_feats
    ef = edge_feats
    src2 = edge_index[0].reshape(NW, CPW, CH)
    dst2 = edge_index[1].reshape(NW, CPW, CH)
    src4 = edge_index[0].reshape(NW, NG, CPG, CH)
    dst4 = edge_index[1].reshape(NW, NG, CPG, CH)

    pm = params["msg"]
    pad = ((0, 0), (0, D - R3))
    a1 = jnp.pad(pm["A"][0:D], pad)
    a2 = jnp.pad(pm["A"][D:2 * D], pad)
    a3 = jnp.pad(pm["A"][2 * D:3 * D], pad)
    bm = jnp.pad(pm["B"], ((0, D - R3), (0, 0)))
    bmsg = pm["b"].reshape(1, D)
    wea = params["We"]["A"]
    web = params["We"]["B"]
    bwe = params["We"]["b"].reshape(1, D)
    x2a = params["x2"]["A"]
    x2b = params["x2"]["B"]
    bx2 = params["x2"]["b"].reshape(1, D)
    ra1 = jnp.stack([blk[0]["A"] for blk in params["res"]])
    rb1 = jnp.stack([blk[0]["B"] for blk in params["res"]])
    rc1 = jnp.stack([blk[0]["b"].reshape(1, D) for blk in params["res"]])
    ra2 = jnp.stack([blk[1]["A"] for blk in params["res"]])
    rb2 = jnp.stack([blk[1]["B"] for blk in params["res"]])
    rc2 = jnp.stack([blk[1]["b"].reshape(1, D) for blk in params["res"]])
    lng_n = params["ln_node"]["g"].reshape(1, D)
    lnb_n = params["ln_node"]["b"].reshape(1, D)
    lng_e = params["ln_edge"]["g"].reshape(1, D)
    lnb_e = params["ln_edge"]["b"].reshape(1, D)

    BA = 1000
    bf16 = jnp.bfloat16
    hd, hs = pl.pallas_call(
        _a_body,
        grid=(N // BA,),
        in_specs=[
            pl.BlockSpec((BA, D), lambda i: (i, 0)),
            pl.BlockSpec((D, D), lambda i: (0, 0)),
            pl.BlockSpec((D, D), lambda i: (0, 0)),
        ],
        out_specs=[
            pl.BlockSpec((BA, D), lambda i: (i, 0)),
            pl.BlockSpec((BA, D), lambda i: (i, 0)),
        ],
        out_shape=[
            jax.ShapeDtypeStruct((N, D), f32),
            jax.ShapeDtypeStruct((N, D), f32),
        ],
    )(h, a1, a2)

    mesh = plsc.VectorSubcoreMesh(core_axis_name="c", subcore_axis_name="s")
    gd, gs = pl.kernel(
        _gather_body,
        out_type=[
            jax.ShapeDtypeStruct((E, D), f32),
            jax.ShapeDtypeStruct((E, D), f32),
        ],
        mesh=mesh,
        scratch_types=[
            pltpu.VMEM((CPG, CH), jnp.int32),
            pltpu.VMEM((CPG, CH), jnp.int32),
            pltpu.VMEM((KG * CH, D), f32),
            pltpu.VMEM((KG * CH, D), f32),
            pltpu.SemaphoreType.DMA,
            pltpu.SemaphoreType.DMA,
        ],
        compiler_params=pltpu.CompilerParams(use_tc_tiling_on_sc=True),
    )(hd, hs, dst4, src4)

    BE = 1280
    mc, e_out = pl.pallas_call(
        _c_body,
        grid=(E // BE,),
        in_specs=[
            pl.BlockSpec((BE, D), lambda i: (i, 0)),
            pl.BlockSpec((BE, D), lambda i: (i, 0)),
            pl.BlockSpec((BE, D), lambda i: (i, 0)),
            pl.BlockSpec((D, D), lambda i: (0, 0)),
            pl.BlockSpec((D, D), lambda i: (0, 0)),
            pl.BlockSpec((1, D), lambda i: (0, 0)),
            pl.BlockSpec((D, R), lambda i: (0, 0)),
            pl.BlockSpec((R, D), lambda i: (0, 0)),
            pl.BlockSpec((1, D), lambda i: (0, 0)),
            pl.BlockSpec((D, R), lambda i: (0, 0)),
            pl.BlockSpec((1, D), lambda i: (0, 0)),
            pl.BlockSpec((1, D), lambda i: (0, 0)),
        ],
        out_specs=[
            pl.BlockSpec((BE, R), lambda i: (i, 0)),
            pl.BlockSpec((BE, D), lambda i: (i, 0)),
        ],
        out_shape=[
            jax.ShapeDtypeStruct((E, R), f32),
            jax.ShapeDtypeStruct((E, D), f32),
        ],
    )(ef, gd, gs, a3, bm, bmsg, wea, web, bwe, x2a, lng_e, lnb_e)

    zeros = jnp.zeros((NPAD, R), f32)
    partials = pl.kernel(
        _scatter_body,
        out_type=jax.ShapeDtypeStruct((NC, NPAD, R), f32),
        mesh=plsc.VectorSubcoreMesh(core_axis_name="c", subcore_axis_name="s"),
        scratch_types=[
            pltpu.VMEM((CPW, CH), jnp.int32),
            pltpu.VMEM((25 * CH, R), f32),
            pltpu.VMEM((NPT, R), f32),
            pltpu.VMEM_SHARED((NPAD, R), f32),
            pltpu.SemaphoreType.DMA,
            pltpu.SemaphoreType.DMA,
        ],
        compiler_params=pltpu.CompilerParams(use_tc_tiling_on_sc=False),
    )(mc, dst2, zeros)
    p0 = partials[0, :N]
    p1 = partials[1, :N]

    BN = 1000
    x_out = pl.pallas_call(
        _e_body,
        grid=(N // BN,),
        in_specs=[
            pl.BlockSpec((BN, D), lambda i: (i, 0)),
            pl.BlockSpec((BN, R), lambda i: (i, 0)),
            pl.BlockSpec((BN, R), lambda i: (i, 0)),
            pl.BlockSpec((D, R), lambda i: (0, 0)),
            pl.BlockSpec((R, D), lambda i: (0, 0)),
            pl.BlockSpec((1, D), lambda i: (0, 0)),
            pl.BlockSpec((3, D, R), lambda i: (0, 0, 0)),
            pl.BlockSpec((3, R, D), lambda i: (0, 0, 0)),
            pl.BlockSpec((3, 1, D), lambda i: (0, 0, 0)),
            pl.BlockSpec((3, D, R), lambda i: (0, 0, 0)),
            pl.BlockSpec((3, R, D), lambda i: (0, 0, 0)),
            pl.BlockSpec((3, 1, D), lambda i: (0, 0, 0)),
            pl.BlockSpec((1, D), lambda i: (0, 0)),
            pl.BlockSpec((1, D), lambda i: (0, 0)),
        ],
        out_specs=pl.BlockSpec((BN, D), lambda i: (i, 0)),
        out_shape=jax.ShapeDtypeStruct((N, D), f32),
    )(h, p0, p1, x2a, x2b, bx2,
      ra1, rb1, rc1, ra2, rb2, rc2, lng_n, lnb_n)

    return (x_out, e_out)


# speedup vs baseline: 1.1539x; 1.1539x over previous
import functools

import jax
import jax.numpy as jnp
from jax import lax
from jax.experimental import pallas as pl
from jax.experimental.pallas import tpu as pltpu
from jax.experimental.pallas import tpu_sc as plsc

N, E, D = 10000, 320000, 128
R3 = 96
R = 32

NC, NS = 2, 16
NW = NC * NS
EPW = E // NW
CH = 80
CPW = EPW // CH
KG = 5
NPAD = 10240
NPT = NPAD // NS


def _silu(x):
    return x * jax.nn.sigmoid(x)


def _a_body(h_ref, a1_ref, a2_ref, hd_ref, hs_ref):
    h = h_ref[...]
    hd_ref[...] = jnp.dot(h, a1_ref[...], preferred_element_type=jnp.float32)
    hs_ref[...] = jnp.dot(h, a2_ref[...], preferred_element_type=jnp.float32)


NG = 5
CPG = CPW // NG


def _gather_body(hd, hs, dst4, src4, gd, gs,
                 idxd_v, idxs_v, bufd, bufs, semg, semw):
    c = lax.axis_index("c")
    s = lax.axis_index("s")
    w = s * NC + c

    def group(g, carry0):
        pltpu.sync_copy(dst4.at[w, g], idxd_v)
        pltpu.sync_copy(src4.at[w, g], idxs_v)

        def super_iter(i, carry):
            c0 = i * KG
            cps = []
            for k in range(KG):
                cps.append(pltpu.async_copy(
                    hd.at[idxd_v.at[c0 + k]], bufd.at[pl.ds(k * CH, CH)], semg))
                cps.append(pltpu.async_copy(
                    hs.at[idxs_v.at[c0 + k]], bufs.at[pl.ds(k * CH, CH)], semg))
            for cp in cps:
                cp.wait()
            eb = ((w * NG + g) * CPG + c0) * CH
            cpo1 = pltpu.async_copy(bufd, gd.at[pl.ds(eb, KG * CH)], semw)
            cpo2 = pltpu.async_copy(bufs, gs.at[pl.ds(eb, KG * CH)], semw)
            cpo1.wait()
            cpo2.wait()
            return carry

        lax.fori_loop(0, CPG // KG, super_iter, 0)
        return carry0

    lax.fori_loop(0, NG, group, 0)


def _c_body(ef_ref, gd_ref, gs_ref, a3_ref, bm_ref, bmsg_ref,
            wea_ref, web_ref, bwe_ref, x2a_ref, lng_ref, lnb_ref,
            mc_ref, eo_ref):
    e = ef_ref[...]
    w = jnp.dot(jnp.dot(e, wea_ref[...], preferred_element_type=jnp.float32),
                web_ref[...], preferred_element_type=jnp.float32) + bwe_ref[...]
    t = gd_ref[...] + gs_ref[...] + jnp.dot(
        e, a3_ref[...], preferred_element_type=jnp.float32)
    u = jnp.dot(t, bm_ref[...], preferred_element_type=jnp.float32) + bmsg_ref[...]
    m = _silu(u) * w
    mc_ref[...] = jnp.dot(m, x2a_ref[...], preferred_element_type=jnp.float32)
    v = e + w
    mu = jnp.mean(v, axis=-1, keepdims=True)
    var = jnp.mean((v - mu) ** 2, axis=-1, keepdims=True)
    vn = (v - mu) / jnp.sqrt(var + 1e-5) * lng_ref[...] + lnb_ref[...]
    eo_ref[...] = _silu(vn)


def _scatter_body(mc, dst2, zeros, out,
                  idx_v, rows, zbuf, acc, semr, sema):
    c = lax.axis_index("c")
    s = lax.axis_index("s")
    w = s * NC + c
    row0 = w * CPW
    pltpu.sync_copy(zeros.at[pl.ds(s * NPT, NPT)], zbuf)
    pltpu.sync_copy(zbuf, acc.at[pl.ds(s * NPT, NPT)])
    plsc.subcore_barrier()
    pltpu.sync_copy(dst2.at[w], idx_v)

    KS = 25

    def super_iter(i, carry):
        c0 = i * KS
        eb = (row0 + c0) * CH
        pltpu.sync_copy(mc.at[pl.ds(eb, KS * CH)], rows)
        cps = []
        for k in range(KS):
            cps.append(pltpu.async_copy(
                rows.at[pl.ds(k * CH, CH)], acc.at[idx_v.at[c0 + k]],
                sema, add=True))
        for cp in cps:
            cp.wait()
        return carry

    lax.fori_loop(0, CPW // KS, super_iter, 0)
    plsc.subcore_barrier()
    pltpu.sync_copy(acc.at[pl.ds(s * NPT, NPT)], zbuf)
    pltpu.sync_copy(zbuf, out.at[c, pl.ds(s * NPT, NPT)])


def _e_body(h_ref, p0_ref, p1_ref, x2a_ref, x2b_ref, bx2_ref,
            ra1_ref, rb1_ref, rc1_ref, ra2_ref, rb2_ref, rc2_ref,
            lng_ref, lnb_ref, x_ref):
    hh = h_ref[...]
    z = jnp.dot(hh, x2a_ref[...], preferred_element_type=jnp.float32)
    z = z + p0_ref[...] + p1_ref[...]
    u = jnp.dot(z, x2b_ref[...], preferred_element_type=jnp.float32) + bx2_ref[...]
    x = _silu(u)
    for k in range(3):
        y = _silu(jnp.dot(jnp.dot(x, ra1_ref[k], preferred_element_type=jnp.float32),
                          rb1_ref[k], preferred_element_type=jnp.float32)
                  + rc1_ref[k])
        y = _silu(jnp.dot(jnp.dot(y, ra2_ref[k], preferred_element_type=jnp.float32),
                          rb2_ref[k], preferred_element_type=jnp.float32)
                  + rc2_ref[k])
        x = x + y
    mu = jnp.mean(x, axis=-1, keepdims=True)
    var = jnp.mean((x - mu) ** 2, axis=-1, keepdims=True)
    xn = (x - mu) / jnp.sqrt(var + 1e-5) * lng_ref[...] + lnb_ref[...]
    x_ref[...] = hh + _silu(xn)


def kernel(node_feats, edge_feats, edge_index, params):
    f32 = jnp.float32
    h = node_feats
    ef = edge_feats
    src2 = edge_index[0].reshape(NW, CPW, CH)
    dst2 = edge_index[1].reshape(NW, CPW, CH)
    src4 = edge_index[0].reshape(NW, NG, CPG, CH)
    dst4 = edge_index[1].reshape(NW, NG, CPG, CH)

    pm = params["msg"]
    pad = ((0, 0), (0, D - R3))
    a1 = jnp.pad(pm["A"][0:D], pad)
    a2 = jnp.pad(pm["A"][D:2 * D], pad)
    a3 = jnp.pad(pm["A"][2 * D:3 * D], pad)
    bm = jnp.pad(pm["B"], ((0, D - R3), (0, 0)))
    bmsg = pm["b"].reshape(1, D)
    wea = params["We"]["A"]
    web = params["We"]["B"]
    bwe = params["We"]["b"].reshape(1, D)
    x2a = params["x2"]["A"]
    x2b = params["x2"]["B"]
    bx2 = params["x2"]["b"].reshape(1, D)
    ra1 = jnp.stack([blk[0]["A"] for blk in params["res"]])
    rb1 = jnp.stack([blk[0]["B"] for blk in params["res"]])
    rc1 = jnp.stack([blk[0]["b"].reshape(1, D) for blk in params["res"]])
    ra2 = jnp.stack([blk[1]["A"] for blk in params["res"]])
    rb2 = jnp.stack([blk[1]["B"] for blk in params["res"]])
    rc2 = jnp.stack([blk[1]["b"].reshape(1, D) for blk in params["res"]])
    lng_n = params["ln_node"]["g"].reshape(1, D)
    lnb_n = params["ln_node"]["b"].reshape(1, D)
    lng_e = params["ln_edge"]["g"].reshape(1, D)
    lnb_e = params["ln_edge"]["b"].reshape(1, D)

    BA = 1000
    bf16 = jnp.bfloat16
    hd, hs = pl.pallas_call(
        _a_body,
        grid=(N // BA,),
        in_specs=[
            pl.BlockSpec((BA, D), lambda i: (i, 0)),
            pl.BlockSpec((D, D), lambda i: (0, 0)),
            pl.BlockSpec((D, D), lambda i: (0, 0)),
        ],
        out_specs=[
            pl.BlockSpec((BA, D), lambda i: (i, 0)),
            pl.BlockSpec((BA, D), lambda i: (i, 0)),
        ],
        out_shape=[
            jax.ShapeDtypeStruct((N, D), f32),
            jax.ShapeDtypeStruct((N, D), f32),
        ],
    )(h, a1, a2)

    mesh = plsc.VectorSubcoreMesh(core_axis_name="c", subcore_axis_name="s")
    gd, gs = pl.kernel(
        _gather_body,
        out_type=[
            jax.ShapeDtypeStruct((E, D), f32),
            jax.ShapeDtypeStruct((E, D), f32),
        ],
        mesh=mesh,
        scratch_types=[
            pltpu.VMEM((CPG, CH), jnp.int32),
            pltpu.VMEM((CPG, CH), jnp.int32),
            pltpu.VMEM((KG * CH, D), f32),
            pltpu.VMEM((KG * CH, D), f32),
            pltpu.SemaphoreType.DMA,
            pltpu.SemaphoreType.DMA,
        ],
        compiler_params=pltpu.CompilerParams(use_tc_tiling_on_sc=True),
    )(hd, hs, dst4, src4)

    BE = 4000
    mc, e_out = pl.pallas_call(
        _c_body,
        grid=(E // BE,),
        in_specs=[
            pl.BlockSpec((BE, D), lambda i: (i, 0)),
            pl.BlockSpec((BE, D), lambda i: (i, 0)),
            pl.BlockSpec((BE, D), lambda i: (i, 0)),
            pl.BlockSpec((D, D), lambda i: (0, 0)),
            pl.BlockSpec((D, D), lambda i: (0, 0)),
            pl.BlockSpec((1, D), lambda i: (0, 0)),
            pl.BlockSpec((D, R), lambda i: (0, 0)),
            pl.BlockSpec((R, D), lambda i: (0, 0)),
            pl.BlockSpec((1, D), lambda i: (0, 0)),
            pl.BlockSpec((D, R), lambda i: (0, 0)),
            pl.BlockSpec((1, D), lambda i: (0, 0)),
            pl.BlockSpec((1, D), lambda i: (0, 0)),
        ],
        out_specs=[
            pl.BlockSpec((BE, R), lambda i: (i, 0)),
            pl.BlockSpec((BE, D), lambda i: (i, 0)),
        ],
        out_shape=[
            jax.ShapeDtypeStruct((E, R), f32),
            jax.ShapeDtypeStruct((E, D), f32),
        ],
    )(ef, gd, gs, a3, bm, bmsg, wea, web, bwe, x2a, lng_e, lnb_e)

    zeros = jnp.zeros((NPAD, R), f32)
    partials = pl.kernel(
        _scatter_body,
        out_type=jax.ShapeDtypeStruct((NC, NPAD, R), f32),
        mesh=plsc.VectorSubcoreMesh(core_axis_name="c", subcore_axis_name="s"),
        scratch_types=[
            pltpu.VMEM((CPW, CH), jnp.int32),
            pltpu.VMEM((25 * CH, R), f32),
            pltpu.VMEM((NPT, R), f32),
            pltpu.VMEM_SHARED((NPAD, R), f32),
            pltpu.SemaphoreType.DMA,
            pltpu.SemaphoreType.DMA,
        ],
        compiler_params=pltpu.CompilerParams(use_tc_tiling_on_sc=False),
    )(mc, dst2, zeros)
    p0 = partials[0, :N]
    p1 = partials[1, :N]

    BN = 1000
    x_out = pl.pallas_call(
        _e_body,
        grid=(N // BN,),
        in_specs=[
            pl.BlockSpec((BN, D), lambda i: (i, 0)),
            pl.BlockSpec((BN, R), lambda i: (i, 0)),
            pl.BlockSpec((BN, R), lambda i: (i, 0)),
            pl.BlockSpec((D, R), lambda i: (0, 0)),
            pl.BlockSpec((R, D), lambda i: (0, 0)),
            pl.BlockSpec((1, D), lambda i: (0, 0)),
            pl.BlockSpec((3, D, R), lambda i: (0, 0, 0)),
            pl.BlockSpec((3, R, D), lambda i: (0, 0, 0)),
            pl.BlockSpec((3, 1, D), lambda i: (0, 0, 0)),
            pl.BlockSpec((3, D, R), lambda i: (0, 0, 0)),
            pl.BlockSpec((3, R, D), lambda i: (0, 0, 0)),
            pl.BlockSpec((3, 1, D), lambda i: (0, 0, 0)),
            pl.BlockSpec((1, D), lambda i: (0, 0)),
            pl.BlockSpec((1, D), lambda i: (0, 0)),
        ],
        out_specs=pl.BlockSpec((BN, D), lambda i: (i, 0)),
        out_shape=jax.ShapeDtypeStruct((N, D), f32),
    )(h, p0, p1, x2a, x2b, bx2,
      ra1, rb1, rc1, ra2, rb2, rc2, lng_n, lnb_n)

    return (x_out, e_out)
